# baseline (speedup 1.0000x reference)
import jax
import jax.numpy as jnp
from jax.experimental import pallas as pl
from jax.experimental.pallas import tpu as pltpu

_N = 100000
_GRID = 8
_BLK = 12544
_TAIL = _N - (_GRID - 1) * _BLK


def _acos(x):
    ax = jnp.abs(x)
    p = jnp.float32(-0.0012624911)
    for c in (0.0066700901, -0.0170881256, 0.0308918810, -0.0501743046,
              0.0889789874, -0.2145988016, 1.5707963050):
        p = p * ax + jnp.float32(c)
    r = jnp.sqrt(jnp.maximum(1.0 - ax, 0.0)) * p
    return jnp.where(x >= 0, r, jnp.float32(jnp.pi) - r)


def _body(t_ref, c_ref, r_ref, v_ref, p_ref, out_ref, acc_ref):
    f32 = jnp.float32
    step = pl.program_id(0)

    ji = jax.lax.broadcasted_iota(jnp.int32, (6, 3), 0)
    ai = jax.lax.broadcasted_iota(jnp.int32, (6, 3), 1)
    M = jnp.zeros((6, 3), f32)
    for h, est_ref in enumerate((c_ref, r_ref)):
        for b in range(3):
            for a in range(3):
                M = jnp.where((ji == 3 * h + b) & (ai == a),
                              est_ref[b, a] - t_ref[b, a], M)
    jb = jax.lax.broadcasted_iota(jnp.int32, (6, 1), 0)
    bias = jnp.zeros((6, 1), f32)
    for h, est_ref in enumerate((c_ref, r_ref)):
        for b in range(3):
            bias = jnp.where(jb == 3 * h + b, est_ref[b, 3] - t_ref[b, 3],
                             bias)
    qg = jax.lax.broadcasted_iota(jnp.int32, (2, 6), 0)
    jg = jax.lax.broadcasted_iota(jnp.int32, (2, 6), 1)
    S = ((jg // 3) == qg).astype(f32)

    P = p_ref[...]
    Y = jnp.dot(M, P, preferred_element_type=f32) + bias
    R = jnp.sqrt(jnp.dot(S, Y * Y, preferred_element_type=f32))

    @pl.when(step == 0)
    def _init():
        acc_ref[...] = jnp.zeros_like(acc_ref)

    @pl.when(step < _GRID - 1)
    def _accum():
        acc_ref[...] += R

    @pl.when(step == _GRID - 1)
    def _fini():
        lane = jax.lax.broadcasted_iota(jnp.int32, (2, _BLK), 1)
        acc_ref[...] += jnp.where(lane < _TAIL, R, 0.0)
        sums = jnp.sum(acc_ref[...], axis=1)
        rmse_c = sums[0] * (1.0 / _N)
        rmse_r = sums[1] * (1.0 / _N)

        deg = f32(180.0 / jnp.pi)

        def scalars(est_ref):
            tr = f32(0.0)
            for b in range(3):
                for a in range(3):
                    tr = tr + est_ref[b, a] * t_ref[b, a]
            x = jnp.clip(0.5 * (tr - 1.0), -1.0, 1.0)
            rre = _acos(x) * deg
            s2 = f32(0.0)
            for b in range(3):
                d = t_ref[b, 3] - est_ref[b, 3]
                s2 = s2 + d * d
            rte = jnp.sqrt(s2)
            recall = jnp.where((rre < 15.0) & (rte < 0.3), f32(1.0), f32(0.0))
            return rre, rte, recall

        rre_c, rte_c, recall_c = scalars(c_ref)
        rre_r, rte_r, recall_r = scalars(r_ref)

        out_ref[0] = rre_c
        out_ref[1] = rte_c
        out_ref[2] = rmse_c
        out_ref[3] = recall_c
        out_ref[4] = rre_r
        out_ref[5] = rte_r
        out_ref[6] = rmse_r
        out_ref[7] = recall_r
        out_ref[8] = v_ref[0]


@jax.jit
def kernel(transform_raw, coarse_trans, refined_trans, src_points, var_rt):
    transform = transform_raw[0]
    planes = src_points.T

    out = pl.pallas_call(
        _body,
        grid=(_GRID,),
        in_specs=[pl.BlockSpec(memory_space=pltpu.SMEM)] * 4
        + [pl.BlockSpec((3, _BLK), lambda i: (0, i))],
        out_specs=pl.BlockSpec(memory_space=pltpu.SMEM),
        out_shape=jax.ShapeDtypeStruct((9,), jnp.float32),
        scratch_shapes=[pltpu.VMEM((2, _BLK), jnp.float32)],
    )(transform, coarse_trans, refined_trans, var_rt, planes)
    return out


# speedup vs baseline: 1.7356x; 1.7356x over previous
import jax
import jax.numpy as jnp
from jax.experimental import pallas as pl
from jax.experimental.pallas import tpu as pltpu

_N = 100000


def _acos(x):
    ax = jnp.abs(x)
    p = jnp.float32(-0.0012624911)
    for c in (0.0066700901, -0.0170881256, 0.0308918810, -0.0501743046,
              0.0889789874, -0.2145988016, 1.5707963050):
        p = p * ax + jnp.float32(c)
    r = jnp.sqrt(jnp.maximum(1.0 - ax, 0.0)) * p
    return jnp.where(x >= 0, r, jnp.float32(jnp.pi) - r)


def _body(t_ref, c_ref, r_ref, v_ref, p_ref, out_ref):
    f32 = jnp.float32

    ji = jax.lax.broadcasted_iota(jnp.int32, (6, 3), 0)
    ai = jax.lax.broadcasted_iota(jnp.int32, (6, 3), 1)
    M = jnp.zeros((6, 3), f32)
    for h, est_ref in enumerate((c_ref, r_ref)):
        for b in range(3):
            for a in range(3):
                M = jnp.where((ji == 3 * h + b) & (ai == a),
                              est_ref[b, a] - t_ref[b, a], M)
    jb = jax.lax.broadcasted_iota(jnp.int32, (6, 1), 0)
    bias = jnp.zeros((6, 1), f32)
    for h, est_ref in enumerate((c_ref, r_ref)):
        for b in range(3):
            bias = jnp.where(jb == 3 * h + b, est_ref[b, 3] - t_ref[b, 3],
                             bias)
    qg = jax.lax.broadcasted_iota(jnp.int32, (2, 6), 0)
    jg = jax.lax.broadcasted_iota(jnp.int32, (2, 6), 1)
    S = ((jg // 3) == qg).astype(f32)

    P = p_ref[...]
    Y = jnp.dot(M, P, preferred_element_type=f32) + bias
    R = jnp.sqrt(jnp.dot(S, Y * Y, preferred_element_type=f32))
    sums = jnp.sum(R, axis=1)
    rmse_c = sums[0] * (1.0 / _N)
    rmse_r = sums[1] * (1.0 / _N)

    deg = f32(180.0 / jnp.pi)

    def scalars(est_ref):
        tr = f32(0.0)
        for b in range(3):
            for a in range(3):
                tr = tr + est_ref[b, a] * t_ref[b, a]
        x = jnp.clip(0.5 * (tr - 1.0), -1.0, 1.0)
        rre = _acos(x) * deg
        s2 = f32(0.0)
        for b in range(3):
            d = t_ref[b, 3] - est_ref[b, 3]
            s2 = s2 + d * d
        rte = jnp.sqrt(s2)
        recall = jnp.where((rre < 15.0) & (rte < 0.3), f32(1.0), f32(0.0))
        return rre, rte, recall

    rre_c, rte_c, recall_c = scalars(c_ref)
    rre_r, rte_r, recall_r = scalars(r_ref)

    out_ref[0] = rre_c
    out_ref[1] = rte_c
    out_ref[2] = rmse_c
    out_ref[3] = recall_c
    out_ref[4] = rre_r
    out_ref[5] = rte_r
    out_ref[6] = rmse_r
    out_ref[7] = recall_r
    out_ref[8] = v_ref[0]


@jax.jit
def kernel(transform_raw, coarse_trans, refined_trans, src_points, var_rt):
    transform = transform_raw[0]
    planes = src_points.T

    out = pl.pallas_call(
        _body,
        in_specs=[pl.BlockSpec(memory_space=pltpu.SMEM)] * 4
        + [pl.BlockSpec(memory_space=pltpu.VMEM)],
        out_specs=pl.BlockSpec(memory_space=pltpu.SMEM),
        out_shape=jax.ShapeDtypeStruct((9,), jnp.float32),
    )(transform, coarse_trans, refined_trans, var_rt, planes)
    return out
